# baseline (speedup 1.0000x reference)
import functools

import jax
import jax.numpy as jnp
from jax.experimental import pallas as pl
from jax.experimental.pallas import tpu as pltpu

_BN_EPS = 1e-5
_TILES = 8


def _lane_allreduce16(v):
    for k in (8, 16, 32, 64):
        v = v + pltpu.roll(v, k, axis=1)
    return v


def _main_body(batch, x_ref, w1i_ref, w2g_ref, b2_ref, y_ref,
               h_ref, sum_ref, ssq_ref, scale_ref, shift_ref):
    i = pl.program_id(0)
    inv_b = 1.0 / float(batch)

    @pl.when(i < _TILES)
    def _stats_pass():
        @pl.when(i == 0)
        def _():
            sum_ref[...] = jnp.zeros_like(sum_ref)
            ssq_ref[...] = jnp.zeros_like(ssq_ref)

        h = jnp.dot(x_ref[...], w1i_ref[...],
                    preferred_element_type=jnp.float32)
        h_ref[i] = h
        h8 = h.reshape(h.shape[0] // 8, 8, 128)
        sum_ref[...] += jnp.sum(h8, axis=0)
        ssq_ref[...] += jnp.sum(h8 * h8, axis=0)

    @pl.when(i == _TILES)
    def _finalize_stats():
        s = _lane_allreduce16(jnp.sum(sum_ref[...], axis=0, keepdims=True))
        q = _lane_allreduce16(jnp.sum(ssq_ref[...], axis=0, keepdims=True))
        mean = s * inv_b
        var = jnp.maximum(q * inv_b - mean * mean, 0.0)
        scale_ref[...] = jax.lax.rsqrt(var + _BN_EPS)
        shift_ref[...] = -mean * scale_ref[...]

    @pl.when(i >= _TILES)
    def _output_pass():
        hn = jnp.maximum(h_ref[i - _TILES] * scale_ref[...] + shift_ref[...],
                         0.0)
        y_ref[...] = jnp.dot(hn, w2g_ref[...],
                             preferred_element_type=jnp.float32) + b2_ref[0, 0]


def _sigmoid_body(y_ref, o_ref):
    o_ref[...] = jax.nn.sigmoid(y_ref[...])


def kernel(x, w1, b1, w2, b2):
    del b1
    batch = x.shape[0]
    hid = w1.shape[0]
    lanes = 16 * hid

    rows = -(-batch // 16)
    if batch % 16:
        x = jnp.pad(x.reshape(-1), (0, rows * 16 * hid - batch * hid))
    x_r = x.reshape(rows, lanes)
    tile_r = max(16, -(-rows // (_TILES * 16)) * 16)
    rows_p = tile_r * _TILES
    if rows_p != rows:
        x_r = jnp.pad(x_r, ((0, rows_p - rows), (0, 0)))

    eye16 = jnp.eye(16, dtype=jnp.float32)
    w1i = jnp.kron(eye16, w1.T)
    w2g = jnp.kron(eye16, w2)

    y_pre = pl.pallas_call(
        functools.partial(_main_body, batch),
        grid=(2 * _TILES,),
        in_specs=[
            pl.BlockSpec((tile_r, lanes),
                         lambda i: (jnp.minimum(i, _TILES - 1), 0)),
            pl.BlockSpec((lanes, lanes), lambda i: (0, 0)),
            pl.BlockSpec((lanes, 16), lambda i: (0, 0)),
            pl.BlockSpec(memory_space=pltpu.MemorySpace.SMEM),
        ],
        out_specs=pl.BlockSpec((tile_r, 16),
                               lambda i: (jnp.maximum(i - _TILES, 0), 0)),
        out_shape=jax.ShapeDtypeStruct((rows_p, 16), jnp.float32),
        scratch_shapes=[
            pltpu.VMEM((_TILES, tile_r, lanes), jnp.float32),
            pltpu.VMEM((hid, lanes), jnp.float32),
            pltpu.VMEM((hid, lanes), jnp.float32),
            pltpu.VMEM((1, lanes), jnp.float32),
            pltpu.VMEM((1, lanes), jnp.float32),
        ],
        compiler_params=pltpu.CompilerParams(
            dimension_semantics=("arbitrary",),
            vmem_limit_bytes=52 * 1024 * 1024,
        ),
    )(x_r, w1i, w2g, b2)

    yd = y_pre.reshape(rows_p // 8, lanes)
    sig_rows = yd.shape[0] // 2
    out = pl.pallas_call(
        _sigmoid_body,
        grid=(2,),
        in_specs=[pl.BlockSpec((sig_rows, lanes), lambda i: (i, 0))],
        out_specs=pl.BlockSpec((sig_rows, lanes), lambda i: (i, 0)),
        out_shape=jax.ShapeDtypeStruct((rows_p // 8, lanes), jnp.float32),
        compiler_params=pltpu.CompilerParams(
            dimension_semantics=("arbitrary",),
            vmem_limit_bytes=24 * 1024 * 1024,
        ),
    )(yd)

    return out.reshape(-1, 1)[:batch]


# speedup vs baseline: 14.9794x; 14.9794x over previous
import functools

import jax
import jax.numpy as jnp
from jax.experimental import pallas as pl
from jax.experimental.pallas import tpu as pltpu

_BN_EPS = 1e-5
_CHUNKS = 8
_IN_TILES = 4
_OUT_TILES = 4


def _bn_mlp_body(tile_in, tile_c, chunk_b, batch,
                 xT_ref, w1_ref, w2blk_ref, b2_ref,
                 o_ref, h_ref, sum_ref, ssq_ref):
    i = pl.program_id(0)
    inv_b = 1.0 / float(batch)

    @pl.when(i < _IN_TILES)
    def _stats_pass():
        @pl.when(i == 0)
        def _():
            sum_ref[...] = jnp.zeros_like(sum_ref)
            ssq_ref[...] = jnp.zeros_like(ssq_ref)

        h = jnp.dot(w1_ref[...], xT_ref[...],
                    preferred_element_type=jnp.float32)
        h_ref[:, pl.ds(i * tile_in, tile_in)] = h
        sum_ref[...] += jnp.sum(h, axis=1, keepdims=True)
        ssq_ref[...] += jnp.sum(h * h, axis=1, keepdims=True)

    @pl.when(i >= _IN_TILES)
    def _output_pass():
        g = i - _IN_TILES
        mean = sum_ref[...] * inv_b
        var = jnp.maximum(ssq_ref[...] * inv_b - mean * mean, 0.0)
        scale = jax.lax.rsqrt(var + _BN_EPS)
        shift = -mean * scale
        hn = jnp.concatenate(
            [jnp.maximum(
                h_ref[:, pl.ds(s * chunk_b + g * tile_c, tile_c)] * scale
                + shift, 0.0)
             for s in range(_CHUNKS)], axis=0)
        y = jnp.dot(w2blk_ref[...], hn,
                    preferred_element_type=jnp.float32) + b2_ref[0, 0]
        o_ref[...] = jax.nn.sigmoid(y)


def kernel(x, w1, b1, w2, b2):
    del b1
    batch = x.shape[0]
    hid = w1.shape[0]
    xT = x.T

    grain = _CHUNKS * _OUT_TILES * 128
    padded = -(-batch // grain) * grain
    if padded != batch:
        xT = jnp.pad(xT, ((0, 0), (0, padded - batch)))
    chunk_b = padded // _CHUNKS
    tile_in = padded // _IN_TILES
    tile_c = chunk_b // _OUT_TILES

    w2blk = jnp.kron(jnp.eye(_CHUNKS, dtype=jnp.float32), w2.reshape(1, hid))

    body = functools.partial(_bn_mlp_body, tile_in, tile_c, chunk_b, batch)

    out = pl.pallas_call(
        body,
        out_shape=jax.ShapeDtypeStruct((_CHUNKS, chunk_b), jnp.float32),
        grid=(_IN_TILES + _OUT_TILES,),
        in_specs=[
            pl.BlockSpec((hid, tile_in),
                         lambda i: (0, jnp.minimum(i, _IN_TILES - 1))),
            pl.BlockSpec((hid, hid), lambda i: (0, 0)),
            pl.BlockSpec((_CHUNKS, _CHUNKS * hid), lambda i: (0, 0)),
            pl.BlockSpec(memory_space=pltpu.MemorySpace.SMEM),
        ],
        out_specs=pl.BlockSpec((_CHUNKS, tile_c),
                               lambda i: (0, jnp.maximum(i - _IN_TILES, 0))),
        scratch_shapes=[
            pltpu.VMEM((hid, padded), jnp.float32),
            pltpu.VMEM((hid, 1), jnp.float32),
            pltpu.VMEM((hid, 1), jnp.float32),
        ],
        compiler_params=pltpu.CompilerParams(
            dimension_semantics=("arbitrary",),
            vmem_limit_bytes=56 * 1024 * 1024,
        ),
    )(xT, w1, w2blk, b2)

    return out.reshape(padded, 1)[:batch]
